# Initial kernel scaffold; baseline (speedup 1.0000x reference)
import jax
import jax.numpy as jnp
from jax.experimental import pallas as pl


def kernel(h, att_src, att_dst, gat_bias, agg_W, agg_b, agg_att, edge_index0, edge_index1, edge_index2):
    raise NotImplementedError("write your pallas kernel here")


# speedup vs baseline: 13.0486x; 13.0486x over previous
import jax
import jax.numpy as jnp
from jax import lax
from jax.experimental import pallas as pl
from jax.experimental.pallas import tpu as pltpu
from jax.experimental.pallas import tpu_sc as plsc

N = 10000
E = 160000
D = 128
S = 3

NC = 2
NS = 16
L = 16

N_PAD = 10240
CHUNK = 128
NCH = 40
EPW = CHUNK * NCH
EC = EPW * NC * NS
ROWS_PER_SUB = N_PAD // NS
BN = 1280
GRID = N_PAD // BN


def _k1_body(h_ref, w_ref, o_ref):
    o_ref[...] = jnp.dot(h_ref[...], w_ref[...],
                         preferred_element_type=jnp.float32)


def _alphas(h_pad, att_src, att_dst):
    w = jnp.zeros((D, D), jnp.float32)
    w = w.at[:, 0:S].set(att_src.T).at[:, S:2 * S].set(att_dst.T)
    al = pl.pallas_call(
        _k1_body,
        grid=(GRID,),
        in_specs=[
            pl.BlockSpec((BN, D), lambda i: (i, 0)),
            pl.BlockSpec((D, D), lambda i: (0, 0)),
        ],
        out_specs=pl.BlockSpec((BN, D), lambda i: (i, 0)),
        out_shape=jax.ShapeDtypeStruct((N_PAD, D), jnp.float32),
    )(h_pad, w)
    alsrc = al[:, 0:S].T.reshape(-1)
    aldst = al[:, S:2 * S].T.reshape(-1)
    return alsrc, aldst


def _take16(x, idx):
    return lax.gather(
        x, idx[:, None],
        lax.GatherDimensionNumbers(
            offset_dims=(), collapsed_slice_dims=(0,), start_index_map=(0,)),
        (1,), mode=lax.GatherScatterMode.PROMISE_IN_BOUNDS)


def _sc1_body(als_hbm, ald_hbm, src_hbm, dst_hbm, ex_hbm, den_hbm,
              asv, adv, srcv, dstv, exv, denv):
    cid = lax.axis_index("c")
    sid = lax.axis_index("s")
    zero16 = jnp.zeros((L,), jnp.float32)
    iota = lax.iota(jnp.int32, L)
    ip1 = jnp.minimum(iota + 1, L - 1)
    im1 = jnp.maximum(iota - 1, 0)

    for s in range(S):
        pltpu.sync_copy(als_hbm.at[pl.ds(s * N_PAD, N_PAD)], asv)
        pltpu.sync_copy(ald_hbm.at[pl.ds(s * N_PAD, N_PAD)], adv)
        pltpu.sync_copy(src_hbm.at[s, cid, sid], srcv)
        pltpu.sync_copy(dst_hbm.at[s, cid, sid], dstv)

        def _zd(i, _):
            denv[pl.ds(i * L, L)] = zero16
            return 0
        lax.fori_loop(0, N_PAD // L, _zd, 0)

        def _chunk(ch, _):
            for v in range(CHUNK // L):
                si = srcv[ch, pl.ds(v * L, L)]
                di = dstv[ch, pl.ds(v * L, L)]
                a = plsc.load_gather(asv, [si]) + plsc.load_gather(adv, [di])
                a = jnp.where(a >= 0.0, a, a * jnp.float32(0.01))
                e = jnp.exp(a)
                exv[ch, pl.ds(v * L, L)] = e
                k, vv = plsc.sort_key_val(di, e)
                knext = _take16(k, ip1)
                kprev = _take16(k, im1)
                lasts = (k != knext) | (iota == L - 1)
                firsts = (k != kprev) | (iota == 0)
                c = plsc.cumsum(vv)
                cprev = jnp.where(iota == 0, 0.0, _take16(c, im1))
                plsc.addupdate_scatter(denv, [k], c, mask=lasts)
                plsc.addupdate_scatter(denv, [k], -cprev, mask=firsts)
            return 0
        lax.fori_loop(0, NCH, _chunk, 0)

        wid = (s * NC + cid) * NS + sid
        pltpu.sync_copy(exv, ex_hbm.at[s, cid, sid])
        pltpu.sync_copy(denv, den_hbm.at[pl.ds(wid * N_PAD, N_PAD)])


def _sc2_body(h_hbm, src_hbm, dst_hbm, ex_hbm, num_hbm,
              srcv, dstv, exv, rows, acc, sem):
    cid = lax.axis_index("c")
    sid = lax.axis_index("s")
    zero16 = jnp.zeros((L,), jnp.float32)

    for s in range(S):
        wid = (s * NC + cid) * NS + sid
        pltpu.sync_copy(src_hbm.at[s, cid, sid], srcv)
        pltpu.sync_copy(dst_hbm.at[s, cid, sid], dstv)
        pltpu.sync_copy(ex_hbm.at[pl.ds(wid * EPW, EPW)], exv)

        def _zrow(i, _):
            for j in range(D // L):
                rows[i, pl.ds(j * L, L)] = zero16
            return 0
        lax.fori_loop(0, CHUNK, _zrow, 0)
        for kb in range(ROWS_PER_SUB // CHUNK):
            pltpu.sync_copy(
                rows, acc.at[pl.ds(sid * ROWS_PER_SUB + kb * CHUNK, CHUNK)])
        plsc.subcore_barrier()

        def _chunk(ch, _):
            pltpu.async_copy(h_hbm.at[srcv.at[ch]], rows, sem).wait()

            def _srow(i, _):
                bi = jnp.full((L,), 0, dtype=jnp.int32) + (ch * CHUNK + i)
                b = plsc.load_gather(exv, [bi])
                for j in range(D // L):
                    rows[i, pl.ds(j * L, L)] = rows[i, pl.ds(j * L, L)] * b
                return 0
            lax.fori_loop(0, CHUNK, _srow, 0)
            pltpu.sync_copy(rows, acc.at[dstv.at[ch]], add=True)
            return 0
        lax.fori_loop(0, NCH, _chunk, 0)
        plsc.subcore_barrier()

        pltpu.sync_copy(
            acc.at[pl.ds(sid * ROWS_PER_SUB, ROWS_PER_SUB)],
            num_hbm.at[s, cid, pl.ds(sid * ROWS_PER_SUB, ROWS_PER_SUB)])
        plsc.subcore_barrier()


def _edge_phase(h_pad, alsrc, aldst, srcs, dsts):
    mesh = plsc.VectorSubcoreMesh(core_axis_name="c", subcore_axis_name="s")
    run1 = pl.kernel(
        _sc1_body,
        out_type=[
            pltpu.HBM((S, NC, NS, NCH, CHUNK), jnp.float32),
            pltpu.HBM((S * NC * NS * N_PAD,), jnp.float32),
        ],
        mesh=mesh,
        compiler_params=pltpu.CompilerParams(needs_layout_passes=False),
        scratch_types=[
            pltpu.VMEM((N_PAD,), jnp.float32),
            pltpu.VMEM((N_PAD,), jnp.float32),
            pltpu.VMEM((NCH, CHUNK), jnp.int32),
            pltpu.VMEM((NCH, CHUNK), jnp.int32),
            pltpu.VMEM((NCH, CHUNK), jnp.float32),
            pltpu.VMEM((N_PAD,), jnp.float32),
        ],
    )
    ex, den_flat = run1(alsrc, aldst, srcs, dsts)

    run2 = pl.kernel(
        _sc2_body,
        out_type=pltpu.HBM((S, NC, N_PAD, D), jnp.float32),
        mesh=mesh,
        compiler_params=pltpu.CompilerParams(needs_layout_passes=False),
        scratch_types=[
            pltpu.VMEM((NCH, CHUNK), jnp.int32),
            pltpu.VMEM((NCH, CHUNK), jnp.int32),
            pltpu.VMEM((EPW,), jnp.float32),
            pltpu.VMEM((CHUNK, D), jnp.float32),
            pltpu.VMEM_SHARED((N_PAD, D), jnp.float32),
            pltpu.SemaphoreType.DMA,
        ],
    )
    num = run2(h_pad, srcs, dsts, ex.reshape(-1))
    return num, den_flat


def _k2a_body(num_ref, den_ref, bias_ref, w_ref, b_ref, att_ref,
              emb_ref, sp_ref):
    i = pl.program_id(0)
    lane = lax.broadcasted_iota(jnp.int32, (1, D), 1)
    ridx = lax.broadcasted_iota(jnp.int32, (BN, 1), 0) + i * BN
    valid = (ridx < N).astype(jnp.float32)
    sp_row = jnp.zeros((1, 1, D), jnp.float32)
    for s in range(S):
        num = num_ref[s, 0] + num_ref[s, 1]
        den = jnp.sum(den_ref[s], axis=1, keepdims=True)
        emb = num / (den + jnp.float32(1e-16)) + bias_ref[pl.ds(s, 1), :]
        emb_ref[s] = emb
        proj = jnp.tanh(
            jnp.dot(emb, w_ref[...], preferred_element_type=jnp.float32)
            + b_ref[...])
        spv = jnp.dot(proj, att_ref[...],
                      preferred_element_type=jnp.float32)
        part = jnp.sum(spv * valid)
        sp_row = sp_row + jnp.where(lane == s, part, 0.0)
    sp_ref[...] = sp_row


def _post_a(num, denp, gat_bias, agg_W, agg_b, agg_att):
    return pl.pallas_call(
        _k2a_body,
        grid=(GRID,),
        in_specs=[
            pl.BlockSpec((S, NC, BN, D), lambda i: (0, 0, i, 0)),
            pl.BlockSpec((S, BN, NC * NS), lambda i: (0, i, 0)),
            pl.BlockSpec((S, D), lambda i: (0, 0)),
            pl.BlockSpec((D, D), lambda i: (0, 0)),
            pl.BlockSpec((1, D), lambda i: (0, 0)),
            pl.BlockSpec((D, 1), lambda i: (0, 0)),
        ],
        out_specs=[
            pl.BlockSpec((S, BN, D), lambda i: (0, i, 0)),
            pl.BlockSpec((1, 1, D), lambda i: (i, 0, 0)),
        ],
        out_shape=[
            jax.ShapeDtypeStruct((S, N_PAD, D), jnp.float32),
            jax.ShapeDtypeStruct((GRID, 1, D), jnp.float32),
        ],
    )(num, denp, gat_bias, agg_W, agg_b.reshape(1, D), agg_att.reshape(D, 1))


def _k2b_body(emb_ref, bt_ref, z_ref):
    acc = emb_ref[0] * bt_ref[pl.ds(0, 1), :]
    for s in range(1, S):
        acc = acc + emb_ref[s] * bt_ref[pl.ds(s, 1), :]
    z_ref[...] = acc


def _post_b(emb, betat):
    return pl.pallas_call(
        _k2b_body,
        grid=(GRID,),
        in_specs=[
            pl.BlockSpec((S, BN, D), lambda i: (0, i, 0)),
            pl.BlockSpec((S, D), lambda i: (0, 0)),
        ],
        out_specs=pl.BlockSpec((BN, D), lambda i: (i, 0)),
        out_shape=jax.ShapeDtypeStruct((N_PAD, D), jnp.float32),
    )(emb, betat)


def kernel(h, att_src, att_dst, gat_bias, agg_W, agg_b, agg_att,
           edge_index0, edge_index1, edge_index2):
    h = h.astype(jnp.float32)
    h_pad = jnp.zeros((N_PAD, D), jnp.float32).at[:N].set(h)

    alsrc, aldst = _alphas(h_pad, att_src, att_dst)

    srcs = []
    dsts = []
    for ei in (edge_index0, edge_index1, edge_index2):
        src = jnp.concatenate(
            [ei[0], jnp.zeros((EC - E,), jnp.int32)])
        dst = jnp.concatenate(
            [ei[1], jnp.full((EC - E,), N_PAD - 1, jnp.int32)])
        srcs.append(src.reshape(NC, NS, NCH, CHUNK))
        dsts.append(dst.reshape(NC, NS, NCH, CHUNK))
    srcs = jnp.stack(srcs)
    dsts = jnp.stack(dsts)

    num, den_flat = _edge_phase(h_pad, alsrc, aldst, srcs, dsts)
    denp = den_flat.reshape(S, NC * NS, N_PAD).transpose(0, 2, 1)

    emb, sp_rows = _post_a(num, denp, gat_bias, agg_W, agg_b, agg_att)
    sp = sp_rows.sum(axis=(0, 1))[:S] / jnp.float32(N)
    beta = jax.nn.softmax(sp)
    betat = jnp.broadcast_to(beta[:, None], (S, D))

    z_pad = _post_b(emb, betat)
    return z_pad[:N]


# speedup vs baseline: 16.6916x; 1.2792x over previous
import jax
import jax.numpy as jnp
from jax import lax
from jax.experimental import pallas as pl
from jax.experimental.pallas import tpu as pltpu
from jax.experimental.pallas import tpu_sc as plsc

N = 10000
E = 160000
D = 128
S = 3

NC = 2
NS = 16
L = 16

N_PAD = 10240
CHUNK = 128
NCH = 40
EPW = CHUNK * NCH
EC = EPW * NC * NS
ROWS_PER_SUB = N_PAD // NS
BN = 1280
GRID = N_PAD // BN


def _k1_body(h_ref, w_ref, o_ref):
    o_ref[...] = jnp.dot(h_ref[...], w_ref[...],
                         preferred_element_type=jnp.float32)


def _alphas(h_pad, att_src, att_dst):
    w = jnp.zeros((D, D), jnp.float32)
    w = w.at[:, 0:S].set(att_src.T).at[:, S:2 * S].set(att_dst.T)
    al = pl.pallas_call(
        _k1_body,
        grid=(GRID,),
        in_specs=[
            pl.BlockSpec((BN, D), lambda i: (i, 0)),
            pl.BlockSpec((D, D), lambda i: (0, 0)),
        ],
        out_specs=pl.BlockSpec((BN, D), lambda i: (i, 0)),
        out_shape=jax.ShapeDtypeStruct((N_PAD, D), jnp.float32),
    )(h_pad, w)
    alsrc = al[:, 0:S].T.reshape(-1)
    aldst = al[:, S:2 * S].T.reshape(-1)
    return alsrc, aldst


def _take16(x, idx):
    return lax.gather(
        x, idx[:, None],
        lax.GatherDimensionNumbers(
            offset_dims=(), collapsed_slice_dims=(0,), start_index_map=(0,)),
        (1,), mode=lax.GatherScatterMode.PROMISE_IN_BOUNDS)


def _sc1_body(als_hbm, ald_hbm, src_hbm, dst_hbm, ex_hbm, den_hbm,
              asv, adv, srcv, dstv, exv, denv):
    cid = lax.axis_index("c")
    sid = lax.axis_index("s")
    zero16 = jnp.zeros((L,), jnp.float32)
    iota = lax.iota(jnp.int32, L)
    ip1 = jnp.minimum(iota + 1, L - 1)
    im1 = jnp.maximum(iota - 1, 0)

    for s in range(S):
        pltpu.sync_copy(als_hbm.at[pl.ds(s * N_PAD, N_PAD)], asv)
        pltpu.sync_copy(ald_hbm.at[pl.ds(s * N_PAD, N_PAD)], adv)
        pltpu.sync_copy(src_hbm.at[s, cid, sid], srcv)
        pltpu.sync_copy(dst_hbm.at[s, cid, sid], dstv)

        def _zd(i, _):
            denv[pl.ds(i * L, L)] = zero16
            return 0
        lax.fori_loop(0, N_PAD // L, _zd, 0)

        def _chunk(ch, _):
            for v in range(CHUNK // L):
                si = srcv[ch, pl.ds(v * L, L)]
                di = dstv[ch, pl.ds(v * L, L)]
                a = plsc.load_gather(asv, [si]) + plsc.load_gather(adv, [di])
                a = jnp.where(a >= 0.0, a, a * jnp.float32(0.01))
                e = jnp.exp(a)
                exv[ch, pl.ds(v * L, L)] = e
                k, vv = plsc.sort_key_val(di, e)
                knext = _take16(k, ip1)
                kprev = _take16(k, im1)
                lasts = (k != knext) | (iota == L - 1)
                firsts = (k != kprev) | (iota == 0)
                c = plsc.cumsum(vv)
                cprev = jnp.where(iota == 0, 0.0, _take16(c, im1))
                plsc.addupdate_scatter(denv, [k], c, mask=lasts)
                plsc.addupdate_scatter(denv, [k], -cprev, mask=firsts)
            return 0
        lax.fori_loop(0, NCH, _chunk, 0)

        wid = (s * NC + cid) * NS + sid
        pltpu.sync_copy(exv, ex_hbm.at[s, cid, sid])
        pltpu.sync_copy(denv, den_hbm.at[pl.ds(wid * N_PAD, N_PAD)])


def _sc2_body(h_hbm, src_hbm, dst_hbm, ex_hbm, num_hbm,
              srcv, dstv, exv, rows0, rows1, acc, sem0, sem1):
    cid = lax.axis_index("c")
    sid = lax.axis_index("s")
    zero16 = jnp.zeros((L,), jnp.float32)

    def _scale_scatter(buf, ch):
        @plsc.parallel_loop(0, CHUNK, 1, unroll=2)
        def _srow(i):
            bi = jnp.full((L,), 0, dtype=jnp.int32) + (ch * CHUNK + i)
            b = plsc.load_gather(exv, [bi])
            for j in range(D // L):
                buf[i, pl.ds(j * L, L)] = buf[i, pl.ds(j * L, L)] * b
        pltpu.sync_copy(buf, acc.at[dstv.at[ch]], add=True)

    for s in range(S):
        wid = (s * NC + cid) * NS + sid
        pltpu.sync_copy(src_hbm.at[s, cid, sid], srcv)
        pltpu.sync_copy(dst_hbm.at[s, cid, sid], dstv)
        pltpu.sync_copy(ex_hbm.at[pl.ds(wid * EPW, EPW)], exv)

        def _zrow(i, _):
            for j in range(D // L):
                rows0[i, pl.ds(j * L, L)] = zero16
            return 0
        lax.fori_loop(0, CHUNK, _zrow, 0)
        for kb in range(ROWS_PER_SUB // CHUNK):
            pltpu.sync_copy(
                rows0, acc.at[pl.ds(sid * ROWS_PER_SUB + kb * CHUNK, CHUNK)])
        plsc.subcore_barrier()

        pltpu.async_copy(h_hbm.at[srcv.at[0]], rows0, sem0)

        def _pair(g, _):
            ch0 = 2 * g
            ch1 = ch0 + 1
            pltpu.async_copy(h_hbm.at[srcv.at[ch1]], rows1, sem1)
            pltpu.make_async_copy(h_hbm.at[srcv.at[ch0]], rows0, sem0).wait()
            _scale_scatter(rows0, ch0)

            @pl.when(g < NCH // 2 - 1)
            def _():
                pltpu.async_copy(h_hbm.at[srcv.at[ch0 + 2]], rows0, sem0)

            pltpu.make_async_copy(h_hbm.at[srcv.at[ch1]], rows1, sem1).wait()
            _scale_scatter(rows1, ch1)
            return 0
        lax.fori_loop(0, NCH // 2, _pair, 0)
        plsc.subcore_barrier()

        pltpu.sync_copy(
            acc.at[pl.ds(sid * ROWS_PER_SUB, ROWS_PER_SUB)],
            num_hbm.at[s, cid, pl.ds(sid * ROWS_PER_SUB, ROWS_PER_SUB)])
        plsc.subcore_barrier()


def _edge_phase(h_pad, alsrc, aldst, srcs, dsts):
    mesh = plsc.VectorSubcoreMesh(core_axis_name="c", subcore_axis_name="s")
    run1 = pl.kernel(
        _sc1_body,
        out_type=[
            pltpu.HBM((S, NC, NS, NCH, CHUNK), jnp.float32),
            pltpu.HBM((S * NC * NS * N_PAD,), jnp.float32),
        ],
        mesh=mesh,
        compiler_params=pltpu.CompilerParams(needs_layout_passes=False),
        scratch_types=[
            pltpu.VMEM((N_PAD,), jnp.float32),
            pltpu.VMEM((N_PAD,), jnp.float32),
            pltpu.VMEM((NCH, CHUNK), jnp.int32),
            pltpu.VMEM((NCH, CHUNK), jnp.int32),
            pltpu.VMEM((NCH, CHUNK), jnp.float32),
            pltpu.VMEM((N_PAD,), jnp.float32),
        ],
    )
    ex, den_flat = run1(alsrc, aldst, srcs, dsts)

    run2 = pl.kernel(
        _sc2_body,
        out_type=pltpu.HBM((S, NC, N_PAD, D), jnp.float32),
        mesh=mesh,
        compiler_params=pltpu.CompilerParams(needs_layout_passes=False),
        scratch_types=[
            pltpu.VMEM((NCH, CHUNK), jnp.int32),
            pltpu.VMEM((NCH, CHUNK), jnp.int32),
            pltpu.VMEM((EPW,), jnp.float32),
            pltpu.VMEM((CHUNK, D), jnp.float32),
            pltpu.VMEM((CHUNK, D), jnp.float32),
            pltpu.VMEM_SHARED((N_PAD, D), jnp.float32),
            pltpu.SemaphoreType.DMA,
            pltpu.SemaphoreType.DMA,
        ],
    )
    num = run2(h_pad, srcs, dsts, ex.reshape(-1))
    return num, den_flat


def _k2a_body(num_ref, den_ref, bias_ref, w_ref, b_ref, att_ref,
              emb_ref, sp_ref):
    i = pl.program_id(0)
    lane = lax.broadcasted_iota(jnp.int32, (1, D), 1)
    ridx = lax.broadcasted_iota(jnp.int32, (BN, 1), 0) + i * BN
    valid = (ridx < N).astype(jnp.float32)
    sp_row = jnp.zeros((1, 1, D), jnp.float32)
    for s in range(S):
        num = num_ref[s, 0] + num_ref[s, 1]
        den = jnp.sum(den_ref[s], axis=1, keepdims=True)
        emb = num / (den + jnp.float32(1e-16)) + bias_ref[pl.ds(s, 1), :]
        emb_ref[s] = emb
        proj = jnp.tanh(
            jnp.dot(emb, w_ref[...], preferred_element_type=jnp.float32)
            + b_ref[...])
        spv = jnp.dot(proj, att_ref[...],
                      preferred_element_type=jnp.float32)
        part = jnp.sum(spv * valid)
        sp_row = sp_row + jnp.where(lane == s, part, 0.0)
    sp_ref[...] = sp_row


def _post_a(num, denp, gat_bias, agg_W, agg_b, agg_att):
    return pl.pallas_call(
        _k2a_body,
        grid=(GRID,),
        in_specs=[
            pl.BlockSpec((S, NC, BN, D), lambda i: (0, 0, i, 0)),
            pl.BlockSpec((S, BN, NC * NS), lambda i: (0, i, 0)),
            pl.BlockSpec((S, D), lambda i: (0, 0)),
            pl.BlockSpec((D, D), lambda i: (0, 0)),
            pl.BlockSpec((1, D), lambda i: (0, 0)),
            pl.BlockSpec((D, 1), lambda i: (0, 0)),
        ],
        out_specs=[
            pl.BlockSpec((S, BN, D), lambda i: (0, i, 0)),
            pl.BlockSpec((1, 1, D), lambda i: (i, 0, 0)),
        ],
        out_shape=[
            jax.ShapeDtypeStruct((S, N_PAD, D), jnp.float32),
            jax.ShapeDtypeStruct((GRID, 1, D), jnp.float32),
        ],
    )(num, denp, gat_bias, agg_W, agg_b.reshape(1, D), agg_att.reshape(D, 1))


def _k2b_body(emb_ref, bt_ref, z_ref):
    acc = emb_ref[0] * bt_ref[pl.ds(0, 1), :]
    for s in range(1, S):
        acc = acc + emb_ref[s] * bt_ref[pl.ds(s, 1), :]
    z_ref[...] = acc


def _post_b(emb, betat):
    return pl.pallas_call(
        _k2b_body,
        grid=(GRID,),
        in_specs=[
            pl.BlockSpec((S, BN, D), lambda i: (0, i, 0)),
            pl.BlockSpec((S, D), lambda i: (0, 0)),
        ],
        out_specs=pl.BlockSpec((BN, D), lambda i: (i, 0)),
        out_shape=jax.ShapeDtypeStruct((N_PAD, D), jnp.float32),
    )(emb, betat)


def kernel(h, att_src, att_dst, gat_bias, agg_W, agg_b, agg_att,
           edge_index0, edge_index1, edge_index2):
    h = h.astype(jnp.float32)
    h_pad = jnp.zeros((N_PAD, D), jnp.float32).at[:N].set(h)

    alsrc, aldst = _alphas(h_pad, att_src, att_dst)

    srcs = []
    dsts = []
    for ei in (edge_index0, edge_index1, edge_index2):
        src = jnp.concatenate(
            [ei[0], jnp.zeros((EC - E,), jnp.int32)])
        dst = jnp.concatenate(
            [ei[1], jnp.full((EC - E,), N_PAD - 1, jnp.int32)])
        srcs.append(src.reshape(NC, NS, NCH, CHUNK))
        dsts.append(dst.reshape(NC, NS, NCH, CHUNK))
    srcs = jnp.stack(srcs)
    dsts = jnp.stack(dsts)

    num, den_flat = _edge_phase(h_pad, alsrc, aldst, srcs, dsts)
    denp = den_flat.reshape(S, NC * NS, N_PAD).transpose(0, 2, 1)

    emb, sp_rows = _post_a(num, denp, gat_bias, agg_W, agg_b, agg_att)
    sp = sp_rows.sum(axis=(0, 1))[:S] / jnp.float32(N)
    beta = jax.nn.softmax(sp)
    betat = jnp.broadcast_to(beta[:, None], (S, D))

    z_pad = _post_b(emb, betat)
    return z_pad[:N]
